# baseline (speedup 1.0000x reference)
import jax
import jax.numpy as jnp
from jax import lax
from jax.experimental import pallas as pl
from jax.experimental.pallas import tpu as pltpu
from jax.experimental.pallas import tpu_sc as plsc

D_VOC = 131
D_MOD = 32
B_TOTAL = 16384 * 200

_info = plsc.get_sparse_core_info()
_NC, _NS = _info.num_cores, _info.num_subcores
_NW = _NC * _NS

_PER_W = B_TOTAL // _NW
_CHUNK = 1024
_STEPS = _PER_W // _CHUNK
_GROUPS = _CHUNK // 16


def _emb_body(tokens_hbm, table_hbm, out_hbm, table_v, tok_v, out_v, _sem):
    wid = lax.axis_index("s") * _NC + lax.axis_index("c")
    w_base = wid * _PER_W

    pltpu.sync_copy(table_hbm, table_v)
    pos0 = lax.iota(jnp.int32, 16)
    dsplats = [jnp.full((16,), d, jnp.int32) for d in range(D_MOD)]

    def chunk(g, carry):
        base = w_base + g * _CHUNK
        pltpu.sync_copy(tokens_hbm.at[pl.ds(base, _CHUNK)], tok_v)

        def group(i, c2):
            tok16 = tok_v[pl.ds(i * 16, 16)]
            pos16 = pos0 + i * 16
            for d in range(D_MOD):
                vals = plsc.load_gather(table_v, [tok16, dsplats[d]])
                plsc.store_scatter(out_v, [pos16, dsplats[d]], vals)
            return c2

        lax.fori_loop(0, _GROUPS, group, 0, unroll=2)
        pltpu.sync_copy(out_v, out_hbm.at[pl.ds(base, _CHUNK)])
        return carry

    lax.fori_loop(0, _STEPS, chunk, 0)


@jax.jit
def _emb_call(tokens_flat, weight):
    mesh = plsc.VectorSubcoreMesh(core_axis_name="c", subcore_axis_name="s")
    f = pl.kernel(
        _emb_body,
        out_type=jax.ShapeDtypeStruct((B_TOTAL, D_MOD), jnp.float32),
        mesh=mesh,
        scratch_types=[
            pltpu.VMEM((D_VOC, D_MOD), jnp.float32),
            pltpu.VMEM((_CHUNK,), jnp.int32),
            pltpu.VMEM((_CHUNK, D_MOD), jnp.float32),
            pltpu.SemaphoreType.DMA,
        ],
        compiler_params=pltpu.CompilerParams(use_tc_tiling_on_sc=False, needs_layout_passes=False),
    )
    return f(tokens_flat, weight)


def kernel(tokens, weight):
    tokens_flat = tokens.reshape(B_TOTAL).astype(jnp.int32)
    out = _emb_call(tokens_flat, weight)
    return out.reshape(*tokens.shape, D_MOD)


# speedup vs baseline: 2.0603x; 2.0603x over previous
import jax
import jax.numpy as jnp
from jax import lax
from jax.experimental import pallas as pl
from jax.experimental.pallas import tpu as pltpu
from jax.experimental.pallas import tpu_sc as plsc

D_VOC = 131
D_MOD = 32
B_TOTAL = 16384 * 200

_info = plsc.get_sparse_core_info()
_NC, _NS = _info.num_cores, _info.num_subcores
_NW = _NC * _NS

_PER_W = B_TOTAL // _NW
_CHUNK = 1024
_STEPS = _PER_W // _CHUNK


def _emb_body(tokens_hbm, table_hbm, out_hbm, table_v, tok_v, out_v, _sem):
    wid = lax.axis_index("s") * _NC + lax.axis_index("c")
    w_base = wid * _PER_W

    pltpu.sync_copy(table_hbm, table_v)
    lane = lax.iota(jnp.int32, 16)
    dlo = lane * D_VOC
    dhi = dlo + 16 * D_VOC

    def chunk(g, carry):
        base = w_base + g * _CHUNK
        pltpu.sync_copy(tokens_hbm.at[pl.ds(base, _CHUNK)], tok_v)

        def group(i, c2):
            tok16 = tok_v[pl.ds(i * 16, 16)]
            for l in range(16):
                tsp = jnp.full((16,), tok16[l], jnp.int32)
                t = i * 16 + l
                out_v[t, pl.ds(0, 16)] = plsc.load_gather(table_v, [dlo + tsp])
                out_v[t, pl.ds(16, 16)] = plsc.load_gather(table_v, [dhi + tsp])
            return c2

        lax.fori_loop(0, _CHUNK // 16, group, 0)
        pltpu.sync_copy(out_v, out_hbm.at[pl.ds(base, _CHUNK)])
        return carry

    lax.fori_loop(0, _STEPS, chunk, 0)


@jax.jit
def _emb_call(tokens_flat, table_t_flat):
    mesh = plsc.VectorSubcoreMesh(core_axis_name="c", subcore_axis_name="s")
    f = pl.kernel(
        _emb_body,
        out_type=jax.ShapeDtypeStruct((B_TOTAL, D_MOD), jnp.float32),
        mesh=mesh,
        scratch_types=[
            pltpu.VMEM((D_MOD * D_VOC,), jnp.float32),
            pltpu.VMEM((_CHUNK,), jnp.int32),
            pltpu.VMEM((_CHUNK, D_MOD), jnp.float32),
            pltpu.SemaphoreType.DMA,
        ],
        compiler_params=pltpu.CompilerParams(
            use_tc_tiling_on_sc=False, needs_layout_passes=False),
    )
    return f(tokens_flat, table_t_flat)


def kernel(tokens, weight):
    tokens_flat = tokens.reshape(B_TOTAL).astype(jnp.int32)
    table_t_flat = weight.T.reshape(-1)
    out = _emb_call(tokens_flat, table_t_flat)
    return out.reshape(*tokens.shape, D_MOD)


# speedup vs baseline: 2.6636x; 1.2928x over previous
import jax
import jax.numpy as jnp
from jax import lax
from jax.experimental import pallas as pl
from jax.experimental.pallas import tpu as pltpu
from jax.experimental.pallas import tpu_sc as plsc

D_VOC = 131
D_MOD = 32
B_TOTAL = 16384 * 200

_info = plsc.get_sparse_core_info()
_NC, _NS = _info.num_cores, _info.num_subcores
_NW = _NC * _NS

_PER_W = B_TOTAL // _NW
_CHUNK = 512
_STEPS = _PER_W // _CHUNK


def _emb_body(tokens_hbm, table_hbm, out_hbm, table_v, tok_v, out_v, _sem):
    wid = lax.axis_index("s") * _NC + lax.axis_index("c")
    w_base = wid * _PER_W

    pltpu.sync_copy(table_hbm, table_v)
    lane = lax.iota(jnp.int32, 16)
    dlo = lane * D_VOC
    dhi = dlo + 16 * D_VOC

    def chunk(g, carry):
        base = w_base + g * _CHUNK
        pltpu.sync_copy(tokens_hbm.at[pl.ds(base, _CHUNK)], tok_v)

        def group(i, c2):
            tok16 = tok_v[pl.ds(i * 16, 16)]
            for l in range(16):
                tsp = jnp.full((16,), tok16[l], jnp.int32)
                t = i * 16 + l
                out_v[t, pl.ds(0, 16)] = plsc.load_gather(table_v, [dlo + tsp])
                out_v[t, pl.ds(16, 16)] = plsc.load_gather(table_v, [dhi + tsp])
            return c2

        lax.fori_loop(0, _CHUNK // 16, group, 0)
        pltpu.sync_copy(out_v, out_hbm.at[pl.ds(base, _CHUNK)])
        return carry

    lax.fori_loop(0, _STEPS, chunk, 0)


@jax.jit
def _emb_call(tokens_flat, table_t_flat):
    mesh = plsc.VectorSubcoreMesh(core_axis_name="c", subcore_axis_name="s")
    f = pl.kernel(
        _emb_body,
        out_type=jax.ShapeDtypeStruct((B_TOTAL, D_MOD), jnp.float32),
        mesh=mesh,
        scratch_types=[
            pltpu.VMEM((D_MOD * D_VOC,), jnp.float32),
            pltpu.VMEM((_CHUNK,), jnp.int32),
            pltpu.VMEM((_CHUNK, D_MOD), jnp.float32),
            pltpu.SemaphoreType.DMA,
        ],
        compiler_params=pltpu.CompilerParams(
            needs_layout_passes=False),
    )
    return f(tokens_flat, table_t_flat)


def kernel(tokens, weight):
    tokens_flat = tokens.reshape(B_TOTAL).astype(jnp.int32)
    table_t_flat = weight.T.reshape(-1)
    out = _emb_call(tokens_flat, table_t_flat)
    return out.reshape(*tokens.shape, D_MOD)
